# baseline (speedup 1.0000x reference)
import functools

import jax
import jax.numpy as jnp
import numpy as np
from jax import lax
from jax.experimental import pallas as pl
from jax.experimental.pallas import tpu as pltpu
from jax.experimental.pallas import tpu_sc as plsc

_B, _N, _D = 64, 1024, 768
_NUM_MASKED = int(0.75 * _N)
_NUM_UNMASKED = _N - _NUM_MASKED
_NW = 32
_GROWS_W = _B * _NUM_UNMASKED // _NW
_CHUNK = 16
_NCHUNK = _GROWS_W // _CHUNK
_NBUF = 8


def _rotl32(x, d):
    return ((x << np.uint32(d)) | (x >> np.uint32(32 - d))).astype(np.uint32)


def _threefry2x32(k0, k1, x0, x1):
    ks = [np.uint32(k0), np.uint32(k1),
          np.uint32(np.uint32(k0) ^ np.uint32(k1) ^ np.uint32(0x1BD11BDA))]
    rot = [(13, 15, 26, 6), (17, 29, 16, 24)]
    x = [(x0 + ks[0]).astype(np.uint32), (x1 + ks[1]).astype(np.uint32)]
    for i in range(5):
        for r in rot[i % 2]:
            x[0] = (x[0] + x[1]).astype(np.uint32)
            x[1] = _rotl32(x[1], r)
            x[1] = (x[1] ^ x[0]).astype(np.uint32)
        x[0] = (x[0] + ks[(i + 1) % 3]).astype(np.uint32)
        x[1] = (x[1] + ks[(i + 2) % 3] + np.uint32(i + 1)).astype(np.uint32)
    return x


@functools.cache
def _host_indices():
    seed = 1
    size = _B * _N
    k0 = np.uint32((seed >> 32) & 0xFFFFFFFF)
    k1 = np.uint32(seed & 0xFFFFFFFF)
    i = np.arange(size, dtype=np.uint64)
    c_hi = (i >> np.uint64(32)).astype(np.uint32)
    c_lo = (i & np.uint64(0xFFFFFFFF)).astype(np.uint32)
    y = _threefry2x32(k0, k1, c_hi, c_lo)
    bits = (y[0] ^ y[1]).reshape(_B, _N)
    noise = ((bits >> np.uint32(9)) | np.uint32(0x3F800000)).view(np.float32) - np.float32(1.0)
    noise = np.maximum(np.float32(0.0), noise)
    perm = np.argsort(noise, axis=1, kind="stable").astype(np.int32)
    masked = perm[:, :_NUM_MASKED].copy()
    unmasked = perm[:, _NUM_MASKED:].copy()
    mask = np.zeros((_B, _N), dtype=np.bool_)
    np.put_along_axis(mask, masked, True, axis=1)
    flat_rows = (unmasked + np.arange(_B, dtype=np.int32)[:, None] * _N).reshape(128, 128)
    return masked, unmasked, mask, flat_rows


def _sc_body(x_ref, fidx_ref, out_ref, fidx_v,
             buf0, buf1, buf2, buf3, buf4, buf5, buf6, buf7,
             gs0, gs1, gs2, gs3, gs4, gs5, gs6, gs7,
             ws0, ws1, ws2, ws3, ws4, ws5, ws6, ws7):
    w = lax.axis_index("s") * 2 + lax.axis_index("c")
    bufs = (buf0, buf1, buf2, buf3, buf4, buf5, buf6, buf7)
    gsems = (gs0, gs1, gs2, gs3, gs4, gs5, gs6, gs7)
    wsems = (ws0, ws1, ws2, ws3, ws4, ws5, ws6, ws7)

    pltpu.sync_copy(fidx_ref.at[pl.ds(w * 4, 4)], fidx_v)

    def _gather(g, b):
        return pltpu.async_copy(
            x_ref.at[fidx_v.at[g // 8, pl.ds((g % 8) * _CHUNK, _CHUNK)]],
            bufs[b], gsems[b])

    def _put(g, b):
        return pltpu.async_copy(
            bufs[b], out_ref.at[pl.ds(w * _GROWS_W + g * _CHUNK, _CHUNK)], wsems[b])

    cps_g = [_gather(b, b) for b in range(_NBUF)]
    cps_w = [None] * _NBUF

    for g in range(_NCHUNK):
        b = g % _NBUF
        cps_g[b].wait()
        cps_w[b] = _put(g, b)
        if g + _NBUF < _NCHUNK:
            cps_w[b].wait()
            cps_g[b] = _gather(g + _NBUF, b)
    for g in range(max(0, _NCHUNK - _NBUF), _NCHUNK):
        cps_w[g % _NBUF].wait()


@functools.cache
def _sc_call():
    mesh = plsc.VectorSubcoreMesh(core_axis_name="c", subcore_axis_name="s")
    return pl.kernel(
        _sc_body,
        mesh=mesh,
        out_type=jax.ShapeDtypeStruct((_B * _NUM_UNMASKED, _D), jnp.float32),
        scratch_types=(
            [pltpu.VMEM((4, 128), jnp.int32)]
            + [pltpu.VMEM((_CHUNK, _D), jnp.float32) for _ in range(_NBUF)]
            + [pltpu.SemaphoreType.DMA for _ in range(2 * _NBUF)]
        ),
    )


def kernel(patch_embeddings):
    masked_np, unmasked_np, mask_np, flat_rows_np = _host_indices()
    fidx_flat = jnp.asarray(flat_rows_np)
    x2d = patch_embeddings.reshape(_B * _N, _D)
    out2d = _sc_call()(x2d, fidx_flat)
    unmasked_patches_only = out2d.reshape(_B, _NUM_UNMASKED, _D)
    bool_mask = jnp.asarray(mask_np)
    masked_indices = jnp.asarray(masked_np)
    unmasked_indices = jnp.asarray(unmasked_np)
    return (unmasked_patches_only, bool_mask, masked_indices, unmasked_indices)


# speedup vs baseline: 1.0461x; 1.0461x over previous
import functools

import jax
import jax.numpy as jnp
import numpy as np
from jax import lax
from jax.experimental import pallas as pl
from jax.experimental.pallas import tpu as pltpu
from jax.experimental.pallas import tpu_sc as plsc

_B, _N, _D = 64, 1024, 768
_NUM_MASKED = int(0.75 * _N)
_NUM_UNMASKED = _N - _NUM_MASKED
_NW = 32
_GROWS_W = _B * _NUM_UNMASKED // _NW
_CHUNK = 32
_NCHUNK = _GROWS_W // _CHUNK
_NBUF = 4


def _rotl32(x, d):
    return ((x << np.uint32(d)) | (x >> np.uint32(32 - d))).astype(np.uint32)


def _threefry2x32(k0, k1, x0, x1):
    ks = [np.uint32(k0), np.uint32(k1),
          np.uint32(np.uint32(k0) ^ np.uint32(k1) ^ np.uint32(0x1BD11BDA))]
    rot = [(13, 15, 26, 6), (17, 29, 16, 24)]
    x = [(x0 + ks[0]).astype(np.uint32), (x1 + ks[1]).astype(np.uint32)]
    for i in range(5):
        for r in rot[i % 2]:
            x[0] = (x[0] + x[1]).astype(np.uint32)
            x[1] = _rotl32(x[1], r)
            x[1] = (x[1] ^ x[0]).astype(np.uint32)
        x[0] = (x[0] + ks[(i + 1) % 3]).astype(np.uint32)
        x[1] = (x[1] + ks[(i + 2) % 3] + np.uint32(i + 1)).astype(np.uint32)
    return x


@functools.cache
def _host_indices():
    seed = 1
    size = _B * _N
    k0 = np.uint32((seed >> 32) & 0xFFFFFFFF)
    k1 = np.uint32(seed & 0xFFFFFFFF)
    i = np.arange(size, dtype=np.uint64)
    c_hi = (i >> np.uint64(32)).astype(np.uint32)
    c_lo = (i & np.uint64(0xFFFFFFFF)).astype(np.uint32)
    y = _threefry2x32(k0, k1, c_hi, c_lo)
    bits = (y[0] ^ y[1]).reshape(_B, _N)
    noise = ((bits >> np.uint32(9)) | np.uint32(0x3F800000)).view(np.float32) - np.float32(1.0)
    noise = np.maximum(np.float32(0.0), noise)
    perm = np.argsort(noise, axis=1, kind="stable").astype(np.int32)
    masked = perm[:, :_NUM_MASKED].copy()
    unmasked = perm[:, _NUM_MASKED:].copy()
    mask = np.zeros((_B, _N), dtype=np.bool_)
    np.put_along_axis(mask, masked, True, axis=1)
    flat_rows = (unmasked + np.arange(_B, dtype=np.int32)[:, None] * _N).reshape(128, 128)
    return masked, unmasked, mask, flat_rows


def _sc_body(x_ref, fidx_ref, out_ref, fidx_v,
             buf0, buf1, buf2, buf3, gs0, gs1, gs2, gs3, ws0, ws1, ws2, ws3):
    w = lax.axis_index("s") * 2 + lax.axis_index("c")
    bufs = (buf0, buf1, buf2, buf3)
    gsems = (gs0, gs1, gs2, gs3)
    wsems = (ws0, ws1, ws2, ws3)

    pltpu.sync_copy(fidx_ref.at[pl.ds(w * 4, 4)], fidx_v)

    def _gather(g, b):
        return pltpu.async_copy(
            x_ref.at[fidx_v.at[g // 4, pl.ds((g % 4) * _CHUNK, _CHUNK)]],
            bufs[b], gsems[b])

    def _put(g, b):
        return pltpu.async_copy(
            bufs[b], out_ref.at[pl.ds(w * _GROWS_W + g * _CHUNK, _CHUNK)], wsems[b])

    cps_g = [_gather(b, b) for b in range(_NBUF)]
    cps_w = [None] * _NBUF

    for g in range(_NCHUNK):
        b = g % _NBUF
        cps_g[b].wait()
        cps_w[b] = _put(g, b)
        if g + _NBUF < _NCHUNK:
            cps_w[b].wait()
            cps_g[b] = _gather(g + _NBUF, b)
    for g in range(max(0, _NCHUNK - _NBUF), _NCHUNK):
        cps_w[g % _NBUF].wait()


@functools.cache
def _sc_call():
    mesh = plsc.VectorSubcoreMesh(core_axis_name="c", subcore_axis_name="s")
    return pl.kernel(
        _sc_body,
        mesh=mesh,
        out_type=jax.ShapeDtypeStruct((_B * _NUM_UNMASKED, _D), jnp.float32),
        scratch_types=(
            [pltpu.VMEM((4, 128), jnp.int32)]
            + [pltpu.VMEM((_CHUNK, _D), jnp.float32) for _ in range(_NBUF)]
            + [pltpu.SemaphoreType.DMA for _ in range(2 * _NBUF)]
        ),
    )


def _tc_copy_body(mask_in, midx_in, uidx_in, mask_out, midx_out, uidx_out):
    mask_out[...] = mask_in[...]
    midx_out[...] = midx_in[...]
    uidx_out[...] = uidx_in[...]


@functools.cache
def _tc_copy_call():
    return pl.pallas_call(
        _tc_copy_body,
        out_shape=[
            jax.ShapeDtypeStruct((_B, _N), jnp.bool_),
            jax.ShapeDtypeStruct((_B, _NUM_MASKED), jnp.int32),
            jax.ShapeDtypeStruct((_B, _NUM_UNMASKED), jnp.int32),
        ],
    )


def kernel(patch_embeddings):
    masked_np, unmasked_np, mask_np, flat_rows_np = _host_indices()
    fidx_flat = jnp.asarray(flat_rows_np)
    x2d = patch_embeddings.reshape(_B * _N, _D)
    out2d = _sc_call()(x2d, fidx_flat)
    unmasked_patches_only = out2d.reshape(_B, _NUM_UNMASKED, _D)
    bool_mask, masked_indices, unmasked_indices = _tc_copy_call()(
        jnp.asarray(mask_np), jnp.asarray(masked_np), jnp.asarray(unmasked_np))
    return (unmasked_patches_only, bool_mask, masked_indices, unmasked_indices)
